# baseline (speedup 1.0000x reference)
import functools

import jax
import jax.numpy as jnp
from jax.experimental import pallas as pl
from jax.experimental.pallas import tpu as pltpu

_EPSILON = 0.05
_T_STATIC = 200
_BLK = 4096
_PK = _BLK // 128
_MB = _BLK * _T_STATIC // 25600


def _controller_kernel(t_ref, tz_ref, x_ref, w_ref, b_ref, u_ref,
                       action_ref, logp_ref):
    t = t_ref[0]
    t_zero = tz_ref[0]
    x = x_ref[...]
    w = w_ref[...]
    logits = jax.lax.dot_general(
        x, w, (((1,), (0,)), ((), ())),
        preferred_element_type=jnp.float32) + b_ref[0]
    logits_pk = logits.reshape(_PK, 128)
    probs_pk = (1.0 - _EPSILON) * jax.nn.sigmoid(logits_pk) + _EPSILON * jnp.float32(0.05)
    action_ref[...] = (u_ref[...] < probs_pk).astype(jnp.float32)
    log_1 = jnp.log(probs_pk)
    log_0 = jnp.log1p(-probs_pk) + t_zero

    y2 = jax.lax.broadcasted_iota(jnp.int32, (_T_STATIC, 128), 0)
    l2 = jax.lax.broadcasted_iota(jnp.int32, (_T_STATIC, 128), 1)
    r0_2 = jax.lax.shift_right_logical(y2 * 20972, 15)
    cut_2 = 200 * (r0_2 + 1) - 128 * y2
    hi_2 = (l2 >= cut_2).astype(jnp.int32)
    j2 = 128 * y2 + l2 - 200 * (r0_2 + hi_2)
    mask_t = (j2 == t)[None]
    mask_lo = (l2 < cut_2)[None]

    z2 = jax.lax.broadcasted_iota(jnp.int32, (128, _T_STATIC), 0)
    r0_t = jax.lax.shift_right_logical(
        jax.lax.broadcasted_iota(jnp.int32, (128, _T_STATIC), 1) * 20972, 15)
    s0 = (z2 == r0_t).astype(jnp.float32)
    s1 = (z2 == r0_t + 1).astype(jnp.float32)

    dn = (((1,), (0,)), ((), ()))
    a0 = jax.lax.dot_general(log_0, s0, dn, preferred_element_type=jnp.float32)
    b0 = jax.lax.dot_general(log_0, s1, dn, preferred_element_type=jnp.float32)
    a1 = jax.lax.dot_general(log_1, s0, dn, preferred_element_type=jnp.float32)
    b1 = jax.lax.dot_general(log_1, s1, dn, preferred_element_type=jnp.float32)

    v0 = jnp.where(mask_lo, a0.reshape(_MB, _T_STATIC, 1), b0.reshape(_MB, _T_STATIC, 1))
    v1 = jnp.where(mask_lo, a1.reshape(_MB, _T_STATIC, 1), b1.reshape(_MB, _T_STATIC, 1))
    logp_ref[...] = jnp.where(mask_t, v1, v0)


def kernel(x, W, b, T, t):
    B = x.shape[0]
    nsteps = B // _BLK
    u = jax.random.uniform(jax.random.key(42), (B, 1), jnp.float32)
    t_arr = jnp.asarray(t, jnp.int32).reshape(1)
    tz_arr = (jnp.asarray(T, jnp.float32) - jnp.float32(_T_STATIC)).reshape(1)
    b_arr = jnp.asarray(b, jnp.float32).reshape(1)
    action_pk, logp3 = pl.pallas_call(
        _controller_kernel,
        grid=(nsteps,),
        in_specs=[
            pl.BlockSpec(memory_space=pltpu.SMEM),
            pl.BlockSpec(memory_space=pltpu.SMEM),
            pl.BlockSpec((_BLK, 128), lambda i: (i, 0)),
            pl.BlockSpec((128, 1), lambda i: (0, 0)),
            pl.BlockSpec(memory_space=pltpu.SMEM),
            pl.BlockSpec((_PK, 128), lambda i: (i, 0)),
        ],
        out_specs=[
            pl.BlockSpec((_PK, 128), lambda i: (i, 0)),
            pl.BlockSpec((_MB, _T_STATIC, 128), lambda i: (i, 0, 0)),
        ],
        out_shape=[
            jax.ShapeDtypeStruct((B // 128, 128), jnp.float32),
            jax.ShapeDtypeStruct((B * _T_STATIC // 25600, _T_STATIC, 128),
                                 jnp.float32),
        ],
    )(t_arr, tz_arr, x, W.T, b_arr, u.reshape(B // 128, 128))
    return (action_pk.reshape(B, 1), logp3.reshape(B, _T_STATIC))


# speedup vs baseline: 2.6643x; 2.6643x over previous
import jax
import jax.numpy as jnp
from jax.experimental import pallas as pl
from jax.experimental.pallas import tpu as pltpu

_EPSILON = 0.05
_T_STATIC = 200
_BLK = 4096
_PK = _BLK // 128


def _controller_kernel(t_ref, tz_ref, x_ref, w_ref, b_ref, u_ref,
                       action_ref, logp_ref):
    t = t_ref[0]
    t_zero = tz_ref[0]
    x = x_ref[...]
    w = w_ref[...]
    logits = jax.lax.dot_general(
        x, w, (((1,), (0,)), ((), ())),
        preferred_element_type=jnp.float32) + b_ref[0]
    logits_pk = logits.reshape(_PK, 128)
    probs_pk = (1.0 - _EPSILON) * jax.nn.sigmoid(logits_pk) + _EPSILON * jnp.float32(0.05)
    action_ref[...] = (u_ref[...] < probs_pk).astype(jnp.float32)
    log_1 = jnp.log(probs_pk).reshape(_BLK, 1)
    log_0 = (jnp.log1p(-probs_pk) + t_zero).reshape(_BLK, 1)
    col = jax.lax.broadcasted_iota(jnp.int32, (_BLK, _T_STATIC), 1)
    logp_ref[...] = jnp.where(col == t, log_1, log_0)


def kernel(x, W, b, T, t):
    B = x.shape[0]
    u = jax.random.uniform(jax.random.key(42), (B, 1), jnp.float32)
    t_arr = jnp.asarray(t, jnp.int32).reshape(1)
    tz_arr = (jnp.asarray(T, jnp.float32) - jnp.float32(_T_STATIC)).reshape(1)
    b_arr = jnp.asarray(b, jnp.float32).reshape(1)
    grid = (B // _BLK,)
    action_pk, log_p = pl.pallas_call(
        _controller_kernel,
        grid=grid,
        in_specs=[
            pl.BlockSpec(memory_space=pltpu.SMEM),
            pl.BlockSpec(memory_space=pltpu.SMEM),
            pl.BlockSpec((_BLK, 128), lambda i: (i, 0)),
            pl.BlockSpec((128, 1), lambda i: (0, 0)),
            pl.BlockSpec(memory_space=pltpu.SMEM),
            pl.BlockSpec((_PK, 128), lambda i: (i, 0)),
        ],
        out_specs=[
            pl.BlockSpec((_PK, 128), lambda i: (i, 0)),
            pl.BlockSpec((_BLK, _T_STATIC), lambda i: (i, 0)),
        ],
        out_shape=[
            jax.ShapeDtypeStruct((B // 128, 128), jnp.float32),
            jax.ShapeDtypeStruct((B, _T_STATIC), jnp.float32),
        ],
    )(t_arr, tz_arr, x, W.T, b_arr, u.reshape(B // 128, 128))
    return (action_pk.reshape(B, 1), log_p)
